# baseline (speedup 1.0000x reference)
import functools

import jax
import jax.numpy as jnp
from jax import lax
from jax.experimental import pallas as pl
from jax.experimental.pallas import tpu as pltpu
from jax.experimental.pallas import tpu_sc as plsc

_B = 1024
_S = 50
_E = 128
_H = 128
_V = 100000

_NC = 2
_NS = 16
_NW = _NC * _NS
_BPW = _B // _NW
_L = 16
_CH = _E // _L
_INV_S = 1.0 / _S

_VB = 2048
_NV = (_V + _VB - 1) // _VB


def _sc_pool_body(ids_hbm, table_hbm, out_hbm, idx_v, rows0, rows1, acc_v,
                  sem0, sem1):
    wid = lax.axis_index("s") * _NC + lax.axis_index("c")
    base = wid * _BPW
    pltpu.sync_copy(ids_hbm.at[pl.ds(base, _BPW)], idx_v)

    def reduce_row(rows_ref, i):
        accs = tuple(rows_ref[0, pl.ds(c * _L, _L)] for c in range(_CH))

        def body(j, accs):
            return tuple(a + rows_ref[j, pl.ds(c * _L, _L)]
                         for c, a in enumerate(accs))

        accs = lax.fori_loop(1, _S, body, accs)
        for c in range(_CH):
            acc_v[i, pl.ds(c * _L, _L)] = accs[c] * _INV_S

    def body2(k, carry):
        i0 = k * 2
        i1 = i0 + 1
        d0 = pltpu.async_copy(table_hbm.at[idx_v.at[i0]], rows0, sem0)
        d1 = pltpu.async_copy(table_hbm.at[idx_v.at[i1]], rows1, sem1)
        d0.wait()
        reduce_row(rows0, i0)
        d1.wait()
        reduce_row(rows1, i1)
        return carry

    lax.fori_loop(0, _BPW // 2, body2, 0)
    pltpu.sync_copy(acc_v, out_hbm.at[pl.ds(base, _BPW)])


_sc_pool = functools.partial(
    pl.kernel,
    out_type=jax.ShapeDtypeStruct((_B, _E), jnp.float32),
    mesh=plsc.VectorSubcoreMesh(core_axis_name="c", subcore_axis_name="s"),
    scratch_types=[
        pltpu.VMEM((_BPW, _S), jnp.int32),
        pltpu.VMEM((_S, _E), jnp.float32),
        pltpu.VMEM((_S, _E), jnp.float32),
        pltpu.VMEM((_BPW, _E), jnp.float32),
        pltpu.SemaphoreType.DMA,
        pltpu.SemaphoreType.DMA,
    ],
)(_sc_pool_body)


def _tc_body(x_ref, w1_ref, b1_ref, w2_ref, b2_ref, out_ref,
             h_ref, m_ref, s_ref):
    p = pl.program_id(0)
    v = pl.program_id(1)

    @pl.when((p == 0) & (v == 0))
    def _init():
        h = lax.dot_general(x_ref[...], w1_ref[...],
                            (((1,), (1,)), ((), ())),
                            preferred_element_type=jnp.float32)
        h = jnp.maximum(h + b1_ref[...], 0.0)
        h_ref[...] = h.astype(jnp.bfloat16)
        m_ref[...] = jnp.full((_B, 1), -1e30, jnp.float32)
        s_ref[...] = jnp.zeros((_B, 1), jnp.float32)

    w2b = w2_ref[...].astype(jnp.bfloat16)
    logits = lax.dot_general(h_ref[...], w2b,
                             (((1,), (1,)), ((), ())),
                             preferred_element_type=jnp.float32)
    logits = logits + b2_ref[...]
    cols = v * _VB + lax.broadcasted_iota(jnp.int32, (1, _VB), 1)
    logits = jnp.where(cols < _V, logits, -1e30)

    @pl.when(p == 0)
    def _stats():
        bm = jnp.max(logits, axis=1, keepdims=True)
        mnew = jnp.maximum(m_ref[...], bm)
        s_ref[...] = (s_ref[...] * jnp.exp(m_ref[...] - mnew)
                      + jnp.sum(jnp.exp(logits - mnew), axis=1, keepdims=True))
        m_ref[...] = mnew

    @pl.when(p == 1)
    def _write():
        out_ref[...] = logits - (m_ref[...] + jnp.log(s_ref[...]))


def _tc_mlp_logsoftmax(x, W1, b1, W2, b2):
    return pl.pallas_call(
        _tc_body,
        grid=(2, _NV),
        in_specs=[
            pl.BlockSpec((_B, _E), lambda p, v: (0, 0)),
            pl.BlockSpec((_H, _E), lambda p, v: (0, 0)),
            pl.BlockSpec((1, _H), lambda p, v: (0, 0)),
            pl.BlockSpec((_VB, _H), lambda p, v: (v, 0)),
            pl.BlockSpec((1, _VB), lambda p, v: (0, v)),
        ],
        out_specs=pl.BlockSpec((_B, _VB),
                               lambda p, v: (0, jnp.where(p == 0, 0, v))),
        out_shape=jax.ShapeDtypeStruct((_B, _V), jnp.float32),
        scratch_shapes=[
            pltpu.VMEM((_B, _H), jnp.bfloat16),
            pltpu.VMEM((_B, 1), jnp.float32),
            pltpu.VMEM((_B, 1), jnp.float32),
        ],
    )(x, W1, b1.reshape(1, _H), W2, b2.reshape(1, _V))


def kernel(input_ids, emb_table, W1, b1, W2, b2):
    x = _sc_pool(input_ids.astype(jnp.int32), emb_table)
    return _tc_mlp_logsoftmax(x, W1, b1, W2, b2)
